# baseline (speedup 1.0000x reference)
import functools

import jax
import jax.numpy as jnp
from jax import lax
from jax.experimental import pallas as pl
from jax.experimental.pallas import tpu as pltpu
from jax.experimental.pallas import tpu_sc as plsc

N = 100000
M = 10000
E = 160000
K = 27
H = 128

NC, NS = 2, 16

CN = 80
CE = 128
CM = 80

f32 = jnp.float32
i32 = jnp.int32


def _mesh():
    return plsc.VectorSubcoreMesh(
        core_axis_name="c", subcore_axis_name="s", num_cores=NC, num_subcores=NS
    )


def _chunks(nchunks, w, body, active=None):
    n = (nchunks - w + NS - 1) // NS
    if active is not None:
        n = jnp.where(active, n, 0)

    def f(i, carry):
        body(w + i * NS)
        return carry
    lax.fori_loop(0, n, f, 0)


def _sc_voxelize(h_F, x_F, p2v, zrow, onesrow):
    @functools.partial(
        pl.kernel,
        out_type=[
            jax.ShapeDtypeStruct((M, H), f32),
            jax.ShapeDtypeStruct((M, H), f32),
            jax.ShapeDtypeStruct((M, H), f32),
            jax.ShapeDtypeStruct((M, H), f32),
        ],
        mesh=_mesh(),
        scratch_types=[
            pltpu.VMEM_SHARED((M, H), f32),
            pltpu.VMEM((CN,), i32),
            pltpu.VMEM((CN, H), f32),
            pltpu.VMEM((CN, H), f32),
            pltpu.SemaphoreType.DMA,
        ],
    )
    def body(h_hbm, x_hbm, p2v_hbm, zrow_hbm, onesrow_hbm,
             hsum_hbm, xsum_hbm, cnt0_hbm, cnt1_hbm,
             acc_sh, idx_v, rows_v, ones_v, sem):
        c = lax.axis_index("c")
        w = lax.axis_index("s")

        pltpu.sync_copy(zrow_hbm, rows_v)
        pltpu.sync_copy(onesrow_hbm, ones_v)

        def zero_chunk(ci):
            pltpu.sync_copy(rows_v, acc_sh.at[pl.ds(ci * CM, CM)])
        _chunks(M // CM, w, zero_chunk)
        plsc.subcore_barrier()

        def scatter_h(ci):
            base = ci * CN
            pltpu.sync_copy(p2v_hbm.at[pl.ds(base, CN)], idx_v)
            pltpu.sync_copy(h_hbm.at[pl.ds(base, CN)], rows_v)
            pltpu.sync_copy(rows_v, acc_sh.at[idx_v], add=True)
        _chunks(N // CN, w, scatter_h, active=c == 0)

        def scatter_x(ci):
            base = ci * CN
            pltpu.sync_copy(p2v_hbm.at[pl.ds(base, CN)], idx_v)
            pltpu.sync_copy(x_hbm.at[pl.ds(base, CN)], rows_v)
            pltpu.sync_copy(rows_v, acc_sh.at[idx_v], add=True)
        _chunks(N // CN, w, scatter_x, active=c == 1)

        plsc.subcore_barrier()

        def dump_h(ci):
            base = ci * CM
            pltpu.sync_copy(acc_sh.at[pl.ds(base, CM)], rows_v)
            pltpu.sync_copy(rows_v, hsum_hbm.at[pl.ds(base, CM)])
        _chunks(M // CM, w, dump_h, active=c == 0)

        def dump_x(ci):
            base = ci * CM
            pltpu.sync_copy(acc_sh.at[pl.ds(base, CM)], rows_v)
            pltpu.sync_copy(rows_v, xsum_hbm.at[pl.ds(base, CM)])
        _chunks(M // CM, w, dump_x, active=c == 1)

        plsc.subcore_barrier()

        pltpu.sync_copy(zrow_hbm, rows_v)

        def zero2_chunk(ci):
            pltpu.sync_copy(rows_v, acc_sh.at[pl.ds(ci * CM, CM)])
        _chunks(M // CM, w, zero2_chunk)
        plsc.subcore_barrier()

        pbase0 = c * (N // 2)

        def scatter_ones(ci):
            base = pbase0 + ci * CN
            pltpu.sync_copy(p2v_hbm.at[pl.ds(base, CN)], idx_v)
            pltpu.sync_copy(ones_v, acc_sh.at[idx_v], add=True)
        _chunks((N // 2) // CN, w, scatter_ones)

        plsc.subcore_barrier()

        def dump_c0(ci):
            base = ci * CM
            pltpu.sync_copy(acc_sh.at[pl.ds(base, CM)], rows_v)
            pltpu.sync_copy(rows_v, cnt0_hbm.at[pl.ds(base, CM)])
        _chunks(M // CM, w, dump_c0, active=c == 0)

        def dump_c1(ci):
            base = ci * CM
            pltpu.sync_copy(acc_sh.at[pl.ds(base, CM)], rows_v)
            pltpu.sync_copy(rows_v, cnt1_hbm.at[pl.ds(base, CM)])
        _chunks(M // CM, w, dump_c1, active=c == 1)

    return body(h_F, x_F, p2v, zrow, onesrow)


def _pipe_pairs(nchunks, tbase, stage, fire, wait, drain):
    stage(0, tbase)
    fire(0)

    def it(j, carry):
        cc = tbase + 2 * j
        stage(1, cc + 1)
        fire(1)
        wait(0)
        drain(0)
        stage(0, cc + 2)
        fire(0)
        wait(1)
        drain(1)
        return carry

    if nchunks % 2:
        lax.fori_loop(0, nchunks // 2, it, 0)
        wait(0)
        drain(0)
    else:
        lax.fori_loop(0, nchunks // 2 - 1, it, 0)
        stage(1, tbase + nchunks - 1)
        fire(1)
        wait(0)
        drain(0)
        wait(1)
        drain(1)


def _once(body, active):
    def f(i, carry):
        body()
        return carry
    lax.fori_loop(0, jnp.where(active, 1, 0), f, 0)


N2 = 102400
CEZ, CEQ = 80, 40


def _sc_edge_pass(split_edges, tv_a, tv_b, eidx, dst, p2v_pad, zrow):
    CE_ = CEQ if split_edges else CEZ
    n_et = 125

    @functools.partial(
        pl.kernel,
        out_type=[
            jax.ShapeDtypeStruct((N2, H), f32),
            jax.ShapeDtypeStruct((N2, H), f32),
            jax.ShapeDtypeStruct((M, H), f32),
            jax.ShapeDtypeStruct((M, H), f32),
        ],
        mesh=_mesh(),
        scratch_types=[
            pltpu.VMEM_SHARED((M, H), f32),
            pltpu.VMEM((CE_,), i32),
            pltpu.VMEM((CE_,), i32),
            pltpu.VMEM((CE_,), i32),
            pltpu.VMEM((CE_,), i32),
            pltpu.VMEM((CE_, H), f32),
            pltpu.VMEM((CE_, H), f32),
            pltpu.VMEM((CN,), i32),
            pltpu.VMEM((CN,), i32),
            pltpu.VMEM((CN, H), f32),
            pltpu.VMEM((CN, H), f32),
            pltpu.SemaphoreType.DMA,
            pltpu.SemaphoreType.DMA,
        ],
    )
    def body(tva_hbm, tvb_hbm, eidx_hbm, dst_hbm, p2v_hbm, zrow_hbm,
             outa_hbm, outb_hbm, voxa_hbm, voxb_hbm,
             acc_sh, eix0_v, eix1_v, dst0_v, dst1_v, rows0_v, rows1_v,
             pix0_v, pix1_v, prow0_v, prow1_v, sem0, sem1):
        c = lax.axis_index("c")
        w = lax.axis_index("s")

        pltpu.sync_copy(zrow_hbm, prow0_v)

        def zero_chunk(ci):
            pltpu.sync_copy(prow0_v, acc_sh.at[pl.ds(ci * CM, CM)])
        _chunks(M // CM, w, zero_chunk)
        plsc.subcore_barrier()

        eix = (eix0_v, eix1_v)
        dstv = (dst0_v, dst1_v)
        rows = (rows0_v, rows1_v)
        sems = (sem0, sem1)

        def e_stage(b, ci):
            base = ci * CE_ if not split_edges else c * (E // 2) + ci * CE_
            pltpu.sync_copy(eidx_hbm.at[pl.ds(base, CE_)], eix[b])
            pltpu.sync_copy(dst_hbm.at[pl.ds(base, CE_)], dstv[b])

        def e_drain(b):
            pltpu.sync_copy(rows[b], acc_sh.at[dstv[b]], add=True)

        def edges(tv_hbm):
            def fire(b):
                pltpu.async_copy(tv_hbm.at[eix[b]], rows[b], sems[b])

            def wait(b):
                pltpu.make_async_copy(tv_hbm.at[eix[b]], rows[b],
                                      sems[b]).wait()
            _pipe_pairs(n_et, w * n_et, e_stage, fire, wait, e_drain)

        if split_edges:
            edges(tva_hbm)
        else:
            _once(lambda: edges(tva_hbm), c == 0)
            _once(lambda: edges(tvb_hbm), c == 1)

        plsc.subcore_barrier()

        def dump_a(ci):
            base = ci * CM
            pltpu.sync_copy(acc_sh.at[pl.ds(base, CM)], prow0_v)
            pltpu.sync_copy(prow0_v, voxa_hbm.at[pl.ds(base, CM)])
        _chunks(M // CM, w, dump_a, active=c == 0)

        def dump_b(ci):
            base = ci * CM
            pltpu.sync_copy(acc_sh.at[pl.ds(base, CM)], prow0_v)
            pltpu.sync_copy(prow0_v, voxb_hbm.at[pl.ds(base, CM)])
        _chunks(M // CM, w, dump_b, active=c == 1)

        plsc.subcore_barrier()

        pix = (pix0_v, pix1_v)
        prow = (prow0_v, prow1_v)

        def p_stage(b, ci):
            pltpu.sync_copy(p2v_hbm.at[pl.ds(ci * CN, CN)], pix[b])

        def points(vox_hbm, out_hbm):
            n_pt = N2 // CN // NS
            tb = w * n_pt

            def fire(b):
                pltpu.async_copy(vox_hbm.at[pix[b]], prow[b], sems[b])

            def wait(b):
                pltpu.make_async_copy(vox_hbm.at[pix[b]], prow[b],
                                      sems[b]).wait()

            def st_fire(b, ci):
                p_stage(b, ci)
                fire(b)

            st_fire(0, tb)

            def it(j, carry):
                cc = tb + 2 * j
                st_fire(1, cc + 1)
                wait(0)
                pltpu.sync_copy(prow[0], out_hbm.at[pl.ds(cc * CN, CN)])
                st_fire(0, cc + 2)
                wait(1)
                pltpu.sync_copy(prow[1], out_hbm.at[pl.ds((cc + 1) * CN, CN)])
                return carry
            lax.fori_loop(0, n_pt // 2 - 1, it, 0)
            last = tb + n_pt - 1
            st_fire(1, last)
            wait(0)
            pltpu.sync_copy(prow[0], out_hbm.at[pl.ds((last - 1) * CN, CN)])
            wait(1)
            pltpu.sync_copy(prow[1], out_hbm.at[pl.ds(last * CN, CN)])

        _once(lambda: points(voxa_hbm, outa_hbm), c == 0)
        _once(lambda: points(voxb_hbm, outb_hbm), c == 1)

    return body(tv_a, tv_b, eidx, dst, p2v_pad, zrow)[:2]


def _sc_voxelize_rh(rh, p2v, zrow):
    @functools.partial(
        pl.kernel,
        out_type=[
            jax.ShapeDtypeStruct((M, H), f32),
            jax.ShapeDtypeStruct((M, H), f32),
        ],
        mesh=_mesh(),
        scratch_types=[
            pltpu.VMEM_SHARED((M, H), f32),
            pltpu.VMEM((CN,), i32),
            pltpu.VMEM((CN, H), f32),
            pltpu.SemaphoreType.DMA,
        ],
    )
    def body(rh_hbm, p2v_hbm, zrow_hbm, sum0_hbm, sum1_hbm,
             acc_sh, idx_v, rows_v, sem):
        c = lax.axis_index("c")
        w = lax.axis_index("s")

        pltpu.sync_copy(zrow_hbm, rows_v)

        def zero_chunk(ci):
            pltpu.sync_copy(rows_v, acc_sh.at[pl.ds(ci * CM, CM)])
        _chunks(M // CM, w, zero_chunk)
        plsc.subcore_barrier()

        pbase0 = c * (N // 2)

        def scatter_chunk(ci):
            base = pbase0 + ci * CN
            pltpu.sync_copy(p2v_hbm.at[pl.ds(base, CN)], idx_v)
            pltpu.sync_copy(rh_hbm.at[pl.ds(base, CN)], rows_v)
            pltpu.sync_copy(rows_v, acc_sh.at[idx_v], add=True)
        _chunks((N // 2) // CN, w, scatter_chunk)

        plsc.subcore_barrier()

        def dump_0(ci):
            base = ci * CM
            pltpu.sync_copy(acc_sh.at[pl.ds(base, CM)], rows_v)
            pltpu.sync_copy(rows_v, sum0_hbm.at[pl.ds(base, CM)])
        _chunks(M // CM, w, dump_0, active=c == 0)

        def dump_1(ci):
            base = ci * CM
            pltpu.sync_copy(acc_sh.at[pl.ds(base, CM)], rows_v)
            pltpu.sync_copy(rows_v, sum1_hbm.at[pl.ds(base, CM)])
        _chunks(M // CM, w, dump_1, active=c == 1)

    return body(rh, p2v, zrow)


MB = 1000
NB = 2000


def _tc_einsum_zr(hsum, xsum, cnt, Wz_c, Wr_c):
    def kern(hs_ref, xs_ref, cnt_ref, wz_ref, wr_ref, tvz_ref, tvr_ref):
        r = 1.0 / jnp.maximum(cnt_ref[:, 0:1], 1.0)
        vhx = jnp.concatenate([hs_ref[...] * r, xs_ref[...] * r], axis=1)
        tvz_ref[0] = jnp.dot(vhx, wz_ref[0], preferred_element_type=f32)
        tvr_ref[0] = jnp.dot(vhx, wr_ref[0], preferred_element_type=f32)

    grid = (M // MB, K)
    return pl.pallas_call(
        kern,
        grid=grid,
        in_specs=[
            pl.BlockSpec((MB, H), lambda m, k: (m, 0)),
            pl.BlockSpec((MB, H), lambda m, k: (m, 0)),
            pl.BlockSpec((MB, 16), lambda m, k: (m, 0)),
            pl.BlockSpec((1, 2 * H, H), lambda m, k: (k, 0, 0)),
            pl.BlockSpec((1, 2 * H, H), lambda m, k: (k, 0, 0)),
        ],
        out_specs=[
            pl.BlockSpec((1, MB, H), lambda m, k: (k, m, 0)),
            pl.BlockSpec((1, MB, H), lambda m, k: (k, m, 0)),
        ],
        out_shape=[
            jax.ShapeDtypeStruct((K, M, H), f32),
            jax.ShapeDtypeStruct((K, M, H), f32),
        ],
    )(hsum, xsum, cnt, Wz_c, Wr_c)


def _tc_einsum_q(rh0, rh1, xsum, cnt, Wq_c):
    def kern(a_ref, b_ref, xs_ref, cnt_ref, wq_ref, tvq_ref):
        r = 1.0 / jnp.maximum(cnt_ref[:, 0:1], 1.0)
        vhx = jnp.concatenate(
            [(a_ref[...] + b_ref[...]) * r, xs_ref[...] * r], axis=1)
        tvq_ref[0] = jnp.dot(vhx, wq_ref[0], preferred_element_type=f32)

    grid = (M // MB, K)
    return pl.pallas_call(
        kern,
        grid=grid,
        in_specs=[
            pl.BlockSpec((MB, H), lambda m, k: (m, 0)),
            pl.BlockSpec((MB, H), lambda m, k: (m, 0)),
            pl.BlockSpec((MB, H), lambda m, k: (m, 0)),
            pl.BlockSpec((MB, 16), lambda m, k: (m, 0)),
            pl.BlockSpec((1, 2 * H, H), lambda m, k: (k, 0, 0)),
        ],
        out_specs=pl.BlockSpec((1, MB, H), lambda m, k: (k, m, 0)),
        out_shape=jax.ShapeDtypeStruct((K, M, H), f32),
    )(rh0, rh1, xsum, cnt, Wq_c)


def _tc_gates(h_F, x_F, ozf, orf, Wz_l, bz_l, Wr_l, br_l, Wq_l, bq_l):
    def kern(h_ref, x_ref, oz_ref, or_ref, wz_ref, bz_ref, wr_ref, br_ref,
             wq_ref, bq_ref, z_ref, rh_ref, lq_ref):
        h = h_ref[...]
        x = x_ref[...]
        lin_z = (jnp.dot(h, wz_ref[:H], preferred_element_type=f32)
                 + jnp.dot(x, wz_ref[H:], preferred_element_type=f32)
                 + bz_ref[...])
        lin_r = (jnp.dot(h, wr_ref[:H], preferred_element_type=f32)
                 + jnp.dot(x, wr_ref[H:], preferred_element_type=f32)
                 + br_ref[...])
        z = jax.nn.sigmoid(oz_ref[...] + lin_z)
        r = jax.nn.sigmoid(or_ref[...] + lin_r)
        rh = r * h
        lq = (jnp.dot(rh, wq_ref[:H], preferred_element_type=f32)
              + jnp.dot(x, wq_ref[H:], preferred_element_type=f32)
              + bq_ref[...])
        z_ref[...] = z
        rh_ref[...] = rh
        lq_ref[...] = lq

    grid = (N // NB,)
    row = pl.BlockSpec((NB, H), lambda n: (n, 0))
    wspec = pl.BlockSpec((2 * H, H), lambda n: (0, 0))
    bspec = pl.BlockSpec((1, H), lambda n: (0, 0))
    return pl.pallas_call(
        kern,
        grid=grid,
        in_specs=[row, row, row, row, wspec, bspec, wspec, bspec, wspec, bspec],
        out_specs=[row, row, row],
        out_shape=[
            jax.ShapeDtypeStruct((N, H), f32),
            jax.ShapeDtypeStruct((N, H), f32),
            jax.ShapeDtypeStruct((N, H), f32),
        ],
    )(h_F, x_F, ozf, orf, Wz_l, bz_l.reshape(1, H), Wr_l, br_l.reshape(1, H),
      Wq_l, bq_l.reshape(1, H))


def _tc_final(h_F, z, q0, q1, lq):
    def kern(h_ref, z_ref, q0_ref, q1_ref, lq_ref, out_ref):
        q = jnp.tanh(q0_ref[...] + q1_ref[...] + lq_ref[...])
        z = z_ref[...]
        out_ref[...] = (1.0 - z) * h_ref[...] + z * q

    grid = (N // NB,)
    row = pl.BlockSpec((NB, H), lambda n: (n, 0))
    return pl.pallas_call(
        kern,
        grid=grid,
        in_specs=[row, row, row, row, row],
        out_specs=row,
        out_shape=jax.ShapeDtypeStruct((N, H), f32),
    )(h_F, z, q0, q1, lq)


def kernel(h_F, x_F, point2voxel, edge_index, edge_kernel,
           Wz_c, Wz_l, bz_l, Wr_c, Wr_l, br_l, Wq_c, Wq_l, bq_l):
    src = edge_index[0]
    dst = edge_index[1]
    eidx = edge_kernel * M + src
    p2v_pad = jnp.concatenate(
        [point2voxel, jnp.zeros((N2 - N,), i32)])

    zrowN = jnp.zeros((CN, H), f32)
    zrowM = jnp.zeros((CM, H), f32)
    onesN = jnp.ones((CN, H), f32)

    hsum, xsum, cnt0, cnt1 = _sc_voxelize(h_F, x_F, point2voxel,
                                          zrowN, onesN)
    cnt = cnt0[:, :16] + cnt1[:, :16]
    tv_z, tv_r = _tc_einsum_zr(hsum, xsum, cnt, Wz_c, Wr_c)
    out_zF, out_rF = _sc_edge_pass(False, tv_z.reshape(K * M, H),
                                   tv_r.reshape(K * M, H),
                                   eidx, dst, p2v_pad, zrowM)
    z, rh, lq = _tc_gates(h_F, x_F, out_zF, out_rF,
                          Wz_l, bz_l, Wr_l, br_l, Wq_l, bq_l)
    rh0, rh1 = _sc_voxelize_rh(rh, point2voxel, zrowN)
    tv_q = _tc_einsum_q(rh0, rh1, xsum, cnt, Wq_c)
    q0, q1 = _sc_edge_pass(True, tv_q.reshape(K * M, H),
                           tv_q.reshape(K * M, H),
                           eidx, dst, p2v_pad, zrowM)
    h_new = _tc_final(h_F, z, q0, q1, lq)
    return h_new


# speedup vs baseline: 1.0454x; 1.0454x over previous
import functools

import jax
import jax.numpy as jnp
from jax import lax
from jax.experimental import pallas as pl
from jax.experimental.pallas import tpu as pltpu
from jax.experimental.pallas import tpu_sc as plsc

N = 100000
M = 10000
E = 160000
K = 27
H = 128

NC, NS = 2, 16

CN = 80
CE = 128
CM = 80

f32 = jnp.float32
i32 = jnp.int32
bf16 = jnp.bfloat16


def _mesh():
    return plsc.VectorSubcoreMesh(
        core_axis_name="c", subcore_axis_name="s", num_cores=NC, num_subcores=NS
    )


def _chunks(nchunks, w, body, active=None):
    n = (nchunks - w + NS - 1) // NS
    if active is not None:
        n = jnp.where(active, n, 0)

    def f(i, carry):
        body(w + i * NS)
        return carry
    lax.fori_loop(0, n, f, 0)


def _sc_voxelize(h_F, x_F, p2v, zrow, onesrow):
    @functools.partial(
        pl.kernel,
        out_type=[
            jax.ShapeDtypeStruct((M, H), f32),
            jax.ShapeDtypeStruct((M, H), f32),
            jax.ShapeDtypeStruct((M, H), f32),
            jax.ShapeDtypeStruct((M, H), f32),
        ],
        mesh=_mesh(),
        scratch_types=[
            pltpu.VMEM_SHARED((M, H), f32),
            pltpu.VMEM((CN,), i32),
            pltpu.VMEM((CN, H), f32),
            pltpu.VMEM((CN, H), f32),
            pltpu.SemaphoreType.DMA,
        ],
    )
    def body(h_hbm, x_hbm, p2v_hbm, zrow_hbm, onesrow_hbm,
             hsum_hbm, xsum_hbm, cnt0_hbm, cnt1_hbm,
             acc_sh, idx_v, rows_v, ones_v, sem):
        c = lax.axis_index("c")
        w = lax.axis_index("s")

        pltpu.sync_copy(zrow_hbm, rows_v)
        pltpu.sync_copy(onesrow_hbm, ones_v)

        def zero_chunk(ci):
            pltpu.sync_copy(rows_v, acc_sh.at[pl.ds(ci * CM, CM)])
        _chunks(M // CM, w, zero_chunk)
        plsc.subcore_barrier()

        def scatter_h(ci):
            base = ci * CN
            pltpu.sync_copy(p2v_hbm.at[pl.ds(base, CN)], idx_v)
            pltpu.sync_copy(h_hbm.at[pl.ds(base, CN)], rows_v)
            pltpu.sync_copy(rows_v, acc_sh.at[idx_v], add=True)
        _chunks(N // CN, w, scatter_h, active=c == 0)

        def scatter_x(ci):
            base = ci * CN
            pltpu.sync_copy(p2v_hbm.at[pl.ds(base, CN)], idx_v)
            pltpu.sync_copy(x_hbm.at[pl.ds(base, CN)], rows_v)
            pltpu.sync_copy(rows_v, acc_sh.at[idx_v], add=True)
        _chunks(N // CN, w, scatter_x, active=c == 1)

        plsc.subcore_barrier()

        def dump_h(ci):
            base = ci * CM
            pltpu.sync_copy(acc_sh.at[pl.ds(base, CM)], rows_v)
            pltpu.sync_copy(rows_v, hsum_hbm.at[pl.ds(base, CM)])
        _chunks(M // CM, w, dump_h, active=c == 0)

        def dump_x(ci):
            base = ci * CM
            pltpu.sync_copy(acc_sh.at[pl.ds(base, CM)], rows_v)
            pltpu.sync_copy(rows_v, xsum_hbm.at[pl.ds(base, CM)])
        _chunks(M // CM, w, dump_x, active=c == 1)

        plsc.subcore_barrier()

        pltpu.sync_copy(zrow_hbm, rows_v)

        def zero2_chunk(ci):
            pltpu.sync_copy(rows_v, acc_sh.at[pl.ds(ci * CM, CM)])
        _chunks(M // CM, w, zero2_chunk)
        plsc.subcore_barrier()

        pbase0 = c * (N // 2)

        def scatter_ones(ci):
            base = pbase0 + ci * CN
            pltpu.sync_copy(p2v_hbm.at[pl.ds(base, CN)], idx_v)
            pltpu.sync_copy(ones_v, acc_sh.at[idx_v], add=True)
        _chunks((N // 2) // CN, w, scatter_ones)

        plsc.subcore_barrier()

        def dump_c0(ci):
            base = ci * CM
            pltpu.sync_copy(acc_sh.at[pl.ds(base, CM)], rows_v)
            pltpu.sync_copy(rows_v, cnt0_hbm.at[pl.ds(base, CM)])
        _chunks(M // CM, w, dump_c0, active=c == 0)

        def dump_c1(ci):
            base = ci * CM
            pltpu.sync_copy(acc_sh.at[pl.ds(base, CM)], rows_v)
            pltpu.sync_copy(rows_v, cnt1_hbm.at[pl.ds(base, CM)])
        _chunks(M // CM, w, dump_c1, active=c == 1)

    return body(h_F, x_F, p2v, zrow, onesrow)


def _sc_edge_pass(split_edges, tv_a, tv_b, eidx, dst, p2v, zrow):
    n_ec = (E // 2 if split_edges else E) // CE

    @functools.partial(
        pl.kernel,
        out_type=[
            jax.ShapeDtypeStruct((N, H), f32),
            jax.ShapeDtypeStruct((N, H), f32),
            jax.ShapeDtypeStruct((M, H), f32),
            jax.ShapeDtypeStruct((M, H), f32),
        ],
        mesh=_mesh(),
        scratch_types=[
            pltpu.VMEM_SHARED((M, H), f32),
            pltpu.VMEM((CE,), i32),
            pltpu.VMEM((CE,), i32),
            pltpu.VMEM((CE, H), f32),
            pltpu.VMEM((CN,), i32),
            pltpu.VMEM((CN, H), f32),
            pltpu.SemaphoreType.DMA,
        ],
    )
    def body(tva_hbm, tvb_hbm, eidx_hbm, dst_hbm, p2v_hbm, zrow_hbm,
             outa_hbm, outb_hbm, voxa_hbm, voxb_hbm,
             acc_sh, eix_v, dst_v, rows_v, pix_v, prow_v, sem):
        c = lax.axis_index("c")
        w = lax.axis_index("s")

        pltpu.sync_copy(zrow_hbm, prow_v)

        def zero_chunk(ci):
            pltpu.sync_copy(prow_v, acc_sh.at[pl.ds(ci * CM, CM)])
        _chunks(M // CM, w, zero_chunk)
        plsc.subcore_barrier()

        ebase0 = c * (E // 2) if split_edges else 0

        def edge_a(ci):
            base = ebase0 + ci * CE
            pltpu.sync_copy(eidx_hbm.at[pl.ds(base, CE)], eix_v)
            pltpu.sync_copy(dst_hbm.at[pl.ds(base, CE)], dst_v)
            pltpu.async_copy(tva_hbm.at[eix_v], rows_v, sem).wait()
            pltpu.sync_copy(rows_v, acc_sh.at[dst_v], add=True)
        _chunks(n_ec, w, edge_a, active=c == 0)

        def edge_b(ci):
            base = ebase0 + ci * CE
            pltpu.sync_copy(eidx_hbm.at[pl.ds(base, CE)], eix_v)
            pltpu.sync_copy(dst_hbm.at[pl.ds(base, CE)], dst_v)
            pltpu.async_copy(tvb_hbm.at[eix_v], rows_v, sem).wait()
            pltpu.sync_copy(rows_v, acc_sh.at[dst_v], add=True)
        _chunks(n_ec, w, edge_b, active=c == 1)

        plsc.subcore_barrier()

        def dump_a(ci):
            base = ci * CM
            pltpu.sync_copy(acc_sh.at[pl.ds(base, CM)], prow_v)
            pltpu.sync_copy(prow_v, voxa_hbm.at[pl.ds(base, CM)])
        _chunks(M // CM, w, dump_a, active=c == 0)

        def dump_b(ci):
            base = ci * CM
            pltpu.sync_copy(acc_sh.at[pl.ds(base, CM)], prow_v)
            pltpu.sync_copy(prow_v, voxb_hbm.at[pl.ds(base, CM)])
        _chunks(M // CM, w, dump_b, active=c == 1)

        plsc.subcore_barrier()

        def point_a(ci):
            base = ci * CN
            pltpu.sync_copy(p2v_hbm.at[pl.ds(base, CN)], pix_v)
            pltpu.async_copy(voxa_hbm.at[pix_v], prow_v, sem).wait()
            pltpu.sync_copy(prow_v, outa_hbm.at[pl.ds(base, CN)])
        _chunks(N // CN, w, point_a, active=c == 0)

        def point_b(ci):
            base = ci * CN
            pltpu.sync_copy(p2v_hbm.at[pl.ds(base, CN)], pix_v)
            pltpu.async_copy(voxb_hbm.at[pix_v], prow_v, sem).wait()
            pltpu.sync_copy(prow_v, outb_hbm.at[pl.ds(base, CN)])
        _chunks(N // CN, w, point_b, active=c == 1)

    return body(tv_a, tv_b, eidx, dst, p2v, zrow)[:2]


def _sc_voxelize_rh(rh, p2v, zrow):
    @functools.partial(
        pl.kernel,
        out_type=[
            jax.ShapeDtypeStruct((M, H), f32),
            jax.ShapeDtypeStruct((M, H), f32),
        ],
        mesh=_mesh(),
        scratch_types=[
            pltpu.VMEM_SHARED((M, H), f32),
            pltpu.VMEM((CN,), i32),
            pltpu.VMEM((CN, H), f32),
            pltpu.SemaphoreType.DMA,
        ],
    )
    def body(rh_hbm, p2v_hbm, zrow_hbm, sum0_hbm, sum1_hbm,
             acc_sh, idx_v, rows_v, sem):
        c = lax.axis_index("c")
        w = lax.axis_index("s")

        pltpu.sync_copy(zrow_hbm, rows_v)

        def zero_chunk(ci):
            pltpu.sync_copy(rows_v, acc_sh.at[pl.ds(ci * CM, CM)])
        _chunks(M // CM, w, zero_chunk)
        plsc.subcore_barrier()

        pbase0 = c * (N // 2)

        def scatter_chunk(ci):
            base = pbase0 + ci * CN
            pltpu.sync_copy(p2v_hbm.at[pl.ds(base, CN)], idx_v)
            pltpu.sync_copy(rh_hbm.at[pl.ds(base, CN)], rows_v)
            pltpu.sync_copy(rows_v, acc_sh.at[idx_v], add=True)
        _chunks((N // 2) // CN, w, scatter_chunk)

        plsc.subcore_barrier()

        def dump_0(ci):
            base = ci * CM
            pltpu.sync_copy(acc_sh.at[pl.ds(base, CM)], rows_v)
            pltpu.sync_copy(rows_v, sum0_hbm.at[pl.ds(base, CM)])
        _chunks(M // CM, w, dump_0, active=c == 0)

        def dump_1(ci):
            base = ci * CM
            pltpu.sync_copy(acc_sh.at[pl.ds(base, CM)], rows_v)
            pltpu.sync_copy(rows_v, sum1_hbm.at[pl.ds(base, CM)])
        _chunks(M // CM, w, dump_1, active=c == 1)

    return body(rh, p2v, zrow)


MB = 1000
NB = 2000


def _tc_einsum_zr(hsum, xsum, cnt, Wz_c, Wr_c):
    def kern(hs_ref, xs_ref, cnt_ref, wz_ref, wr_ref, tvz_ref, tvr_ref):
        r = 1.0 / jnp.maximum(cnt_ref[:, 0:1], 1.0)
        vhx = jnp.concatenate([hs_ref[...] * r, xs_ref[...] * r],
                              axis=1).astype(bf16)
        tvz_ref[0] = jnp.dot(vhx, wz_ref[0].astype(bf16),
                             preferred_element_type=f32)
        tvr_ref[0] = jnp.dot(vhx, wr_ref[0].astype(bf16),
                             preferred_element_type=f32)

    grid = (M // MB, K)
    return pl.pallas_call(
        kern,
        grid=grid,
        in_specs=[
            pl.BlockSpec((MB, H), lambda m, k: (m, 0)),
            pl.BlockSpec((MB, H), lambda m, k: (m, 0)),
            pl.BlockSpec((MB, 16), lambda m, k: (m, 0)),
            pl.BlockSpec((1, 2 * H, H), lambda m, k: (k, 0, 0)),
            pl.BlockSpec((1, 2 * H, H), lambda m, k: (k, 0, 0)),
        ],
        out_specs=[
            pl.BlockSpec((1, MB, H), lambda m, k: (k, m, 0)),
            pl.BlockSpec((1, MB, H), lambda m, k: (k, m, 0)),
        ],
        out_shape=[
            jax.ShapeDtypeStruct((K, M, H), f32),
            jax.ShapeDtypeStruct((K, M, H), f32),
        ],
    )(hsum, xsum, cnt, Wz_c, Wr_c)


def _tc_einsum_q(rh0, rh1, xsum, cnt, Wq_c):
    def kern(a_ref, b_ref, xs_ref, cnt_ref, wq_ref, tvq_ref):
        r = 1.0 / jnp.maximum(cnt_ref[:, 0:1], 1.0)
        vhx = jnp.concatenate(
            [(a_ref[...] + b_ref[...]) * r, xs_ref[...] * r],
            axis=1).astype(bf16)
        tvq_ref[0] = jnp.dot(vhx, wq_ref[0].astype(bf16),
                             preferred_element_type=f32)

    grid = (M // MB, K)
    return pl.pallas_call(
        kern,
        grid=grid,
        in_specs=[
            pl.BlockSpec((MB, H), lambda m, k: (m, 0)),
            pl.BlockSpec((MB, H), lambda m, k: (m, 0)),
            pl.BlockSpec((MB, H), lambda m, k: (m, 0)),
            pl.BlockSpec((MB, 16), lambda m, k: (m, 0)),
            pl.BlockSpec((1, 2 * H, H), lambda m, k: (k, 0, 0)),
        ],
        out_specs=pl.BlockSpec((1, MB, H), lambda m, k: (k, m, 0)),
        out_shape=jax.ShapeDtypeStruct((K, M, H), f32),
    )(rh0, rh1, xsum, cnt, Wq_c)


def _tc_gates(h_F, x_F, ozf, orf, Wz_l, bz_l, Wr_l, br_l, Wq_l, bq_l):
    def kern(h_ref, x_ref, oz_ref, or_ref, wz_ref, bz_ref, wr_ref, br_ref,
             wq_ref, bq_ref, z_ref, rh_ref, lq_ref):
        h = h_ref[...]
        x = x_ref[...]
        hb = h.astype(bf16)
        xb = x.astype(bf16)
        wzb = wz_ref[...].astype(bf16)
        wrb = wr_ref[...].astype(bf16)
        wqb = wq_ref[...].astype(bf16)
        lin_z = (jnp.dot(hb, wzb[:H], preferred_element_type=f32)
                 + jnp.dot(xb, wzb[H:], preferred_element_type=f32)
                 + bz_ref[...])
        lin_r = (jnp.dot(hb, wrb[:H], preferred_element_type=f32)
                 + jnp.dot(xb, wrb[H:], preferred_element_type=f32)
                 + br_ref[...])
        z = jax.nn.sigmoid(oz_ref[...] + lin_z)
        r = jax.nn.sigmoid(or_ref[...] + lin_r)
        rh = r * h
        lq = (jnp.dot(rh.astype(bf16), wqb[:H], preferred_element_type=f32)
              + jnp.dot(xb, wqb[H:], preferred_element_type=f32)
              + bq_ref[...])
        z_ref[...] = z
        rh_ref[...] = rh
        lq_ref[...] = lq

    grid = (N // NB,)
    row = pl.BlockSpec((NB, H), lambda n: (n, 0))
    wspec = pl.BlockSpec((2 * H, H), lambda n: (0, 0))
    bspec = pl.BlockSpec((1, H), lambda n: (0, 0))
    return pl.pallas_call(
        kern,
        grid=grid,
        in_specs=[row, row, row, row, wspec, bspec, wspec, bspec, wspec, bspec],
        out_specs=[row, row, row],
        out_shape=[
            jax.ShapeDtypeStruct((N, H), f32),
            jax.ShapeDtypeStruct((N, H), f32),
            jax.ShapeDtypeStruct((N, H), f32),
        ],
    )(h_F, x_F, ozf, orf, Wz_l, bz_l.reshape(1, H), Wr_l, br_l.reshape(1, H),
      Wq_l, bq_l.reshape(1, H))


def _tc_final(h_F, z, q0, q1, lq):
    def kern(h_ref, z_ref, q0_ref, q1_ref, lq_ref, out_ref):
        q = jnp.tanh(q0_ref[...] + q1_ref[...] + lq_ref[...])
        z = z_ref[...]
        out_ref[...] = (1.0 - z) * h_ref[...] + z * q

    grid = (N // NB,)
    row = pl.BlockSpec((NB, H), lambda n: (n, 0))
    return pl.pallas_call(
        kern,
        grid=grid,
        in_specs=[row, row, row, row, row],
        out_specs=row,
        out_shape=jax.ShapeDtypeStruct((N, H), f32),
    )(h_F, z, q0, q1, lq)


def kernel(h_F, x_F, point2voxel, edge_index, edge_kernel,
           Wz_c, Wz_l, bz_l, Wr_c, Wr_l, br_l, Wq_c, Wq_l, bq_l):
    src = edge_index[0]
    dst = edge_index[1]
    eidx = edge_kernel * M + src

    zrowN = jnp.zeros((CN, H), f32)
    zrowM = jnp.zeros((CM, H), f32)
    onesN = jnp.ones((CN, H), f32)

    hsum, xsum, cnt0, cnt1 = _sc_voxelize(h_F, x_F, point2voxel,
                                          zrowN, onesN)
    cnt = cnt0[:, :16] + cnt1[:, :16]
    tv_z, tv_r = _tc_einsum_zr(hsum, xsum, cnt, Wz_c, Wr_c)
    out_zF, out_rF = _sc_edge_pass(False, tv_z.reshape(K * M, H),
                                   tv_r.reshape(K * M, H),
                                   eidx, dst, point2voxel, zrowM)
    z, rh, lq = _tc_gates(h_F, x_F, out_zF, out_rF,
                          Wz_l, bz_l, Wr_l, br_l, Wq_l, bq_l)
    rh0, rh1 = _sc_voxelize_rh(rh, point2voxel, zrowN)
    tv_q = _tc_einsum_q(rh0, rh1, xsum, cnt, Wq_c)
    q0, q1 = _sc_edge_pass(True, tv_q.reshape(K * M, H),
                           tv_q.reshape(K * M, H),
                           eidx, dst, point2voxel, zrowM)
    h_new = _tc_final(h_F, z, q0, q1, lq)
    return h_new


# speedup vs baseline: 1.1707x; 1.1198x over previous
import functools

import jax
import jax.numpy as jnp
from jax import lax
from jax.experimental import pallas as pl
from jax.experimental.pallas import tpu as pltpu
from jax.experimental.pallas import tpu_sc as plsc

N = 100000
M = 10000
E = 160000
K = 27
H = 128

NC, NS = 2, 16

CN = 80
CE = 128
CM = 80

f32 = jnp.float32
i32 = jnp.int32


def _mesh():
    return plsc.VectorSubcoreMesh(
        core_axis_name="c", subcore_axis_name="s", num_cores=NC, num_subcores=NS
    )


def _chunks(nchunks, w, body, active=None):
    n = (nchunks - w + NS - 1) // NS
    if active is not None:
        n = jnp.where(active, n, 0)

    def f(i, carry):
        body(w + i * NS)
        return carry
    lax.fori_loop(0, n, f, 0)


def _sc_voxelize(h_F, x_F, p2v, zrow, onesrow):
    @functools.partial(
        pl.kernel,
        out_type=[
            jax.ShapeDtypeStruct((M, H), f32),
            jax.ShapeDtypeStruct((M, H), f32),
            jax.ShapeDtypeStruct((M, H), f32),
            jax.ShapeDtypeStruct((M, H), f32),
        ],
        mesh=_mesh(),
        scratch_types=[
            pltpu.VMEM_SHARED((M, H), f32),
            pltpu.VMEM((CN,), i32),
            pltpu.VMEM((CN, H), f32),
            pltpu.VMEM((CN, H), f32),
            pltpu.SemaphoreType.DMA,
        ],
    )
    def body(h_hbm, x_hbm, p2v_hbm, zrow_hbm, onesrow_hbm,
             hsum_hbm, xsum_hbm, cnt0_hbm, cnt1_hbm,
             acc_sh, idx_v, rows_v, ones_v, sem):
        c = lax.axis_index("c")
        w = lax.axis_index("s")

        pltpu.sync_copy(zrow_hbm, rows_v)
        pltpu.sync_copy(onesrow_hbm, ones_v)

        def zero_chunk(ci):
            pltpu.sync_copy(rows_v, acc_sh.at[pl.ds(ci * CM, CM)])
        _chunks(M // CM, w, zero_chunk)
        plsc.subcore_barrier()

        def scatter_h(ci):
            base = ci * CN
            pltpu.sync_copy(p2v_hbm.at[pl.ds(base, CN)], idx_v)
            pltpu.sync_copy(h_hbm.at[pl.ds(base, CN)], rows_v)
            pltpu.sync_copy(rows_v, acc_sh.at[idx_v], add=True)
        _chunks(N // CN, w, scatter_h, active=c == 0)

        def scatter_x(ci):
            base = ci * CN
            pltpu.sync_copy(p2v_hbm.at[pl.ds(base, CN)], idx_v)
            pltpu.sync_copy(x_hbm.at[pl.ds(base, CN)], rows_v)
            pltpu.sync_copy(rows_v, acc_sh.at[idx_v], add=True)
        _chunks(N // CN, w, scatter_x, active=c == 1)

        plsc.subcore_barrier()

        def dump_h(ci):
            base = ci * CM
            pltpu.sync_copy(acc_sh.at[pl.ds(base, CM)], rows_v)
            pltpu.sync_copy(rows_v, hsum_hbm.at[pl.ds(base, CM)])
        _chunks(M // CM, w, dump_h, active=c == 0)

        def dump_x(ci):
            base = ci * CM
            pltpu.sync_copy(acc_sh.at[pl.ds(base, CM)], rows_v)
            pltpu.sync_copy(rows_v, xsum_hbm.at[pl.ds(base, CM)])
        _chunks(M // CM, w, dump_x, active=c == 1)

        plsc.subcore_barrier()

        pltpu.sync_copy(zrow_hbm, rows_v)

        def zero2_chunk(ci):
            pltpu.sync_copy(rows_v, acc_sh.at[pl.ds(ci * CM, CM)])
        _chunks(M // CM, w, zero2_chunk)
        plsc.subcore_barrier()

        pbase0 = c * (N // 2)

        def scatter_ones(ci):
            base = pbase0 + ci * CN
            pltpu.sync_copy(p2v_hbm.at[pl.ds(base, CN)], idx_v)
            pltpu.sync_copy(ones_v, acc_sh.at[idx_v], add=True)
        _chunks((N // 2) // CN, w, scatter_ones)

        plsc.subcore_barrier()

        def dump_c0(ci):
            base = ci * CM
            pltpu.sync_copy(acc_sh.at[pl.ds(base, CM)], rows_v)
            pltpu.sync_copy(rows_v, cnt0_hbm.at[pl.ds(base, CM)])
        _chunks(M // CM, w, dump_c0, active=c == 0)

        def dump_c1(ci):
            base = ci * CM
            pltpu.sync_copy(acc_sh.at[pl.ds(base, CM)], rows_v)
            pltpu.sync_copy(rows_v, cnt1_hbm.at[pl.ds(base, CM)])
        _chunks(M // CM, w, dump_c1, active=c == 1)

    return body(h_F, x_F, p2v, zrow, onesrow)


def _sc_edge_pass(split_edges, tv_a, tv_b, eidx, dst, p2v, zrow):
    n_ec = (E // 2 if split_edges else E) // CE

    @functools.partial(
        pl.kernel,
        out_type=[
            jax.ShapeDtypeStruct((N, H), f32),
            jax.ShapeDtypeStruct((N, H), f32),
            jax.ShapeDtypeStruct((M, H), f32),
            jax.ShapeDtypeStruct((M, H), f32),
        ],
        mesh=_mesh(),
        scratch_types=[
            pltpu.VMEM_SHARED((M, H), f32),
            pltpu.VMEM((CE,), i32),
            pltpu.VMEM((CE,), i32),
            pltpu.VMEM((CE, H), f32),
            pltpu.VMEM((CN,), i32),
            pltpu.VMEM((CN, H), f32),
            pltpu.SemaphoreType.DMA,
        ],
    )
    def body(tva_hbm, tvb_hbm, eidx_hbm, dst_hbm, p2v_hbm, zrow_hbm,
             outa_hbm, outb_hbm, voxa_hbm, voxb_hbm,
             acc_sh, eix_v, dst_v, rows_v, pix_v, prow_v, sem):
        c = lax.axis_index("c")
        w = lax.axis_index("s")

        pltpu.sync_copy(zrow_hbm, prow_v)

        def zero_chunk(ci):
            pltpu.sync_copy(prow_v, acc_sh.at[pl.ds(ci * CM, CM)])
        _chunks(M // CM, w, zero_chunk)
        plsc.subcore_barrier()

        ebase0 = c * (E // 2) if split_edges else 0

        def edge_a(ci):
            base = ebase0 + ci * CE
            pltpu.sync_copy(eidx_hbm.at[pl.ds(base, CE)], eix_v)
            pltpu.sync_copy(dst_hbm.at[pl.ds(base, CE)], dst_v)
            pltpu.async_copy(tva_hbm.at[eix_v], rows_v, sem).wait()
            pltpu.sync_copy(rows_v, acc_sh.at[dst_v], add=True)
        _chunks(n_ec, w, edge_a, active=c == 0)

        def edge_b(ci):
            base = ebase0 + ci * CE
            pltpu.sync_copy(eidx_hbm.at[pl.ds(base, CE)], eix_v)
            pltpu.sync_copy(dst_hbm.at[pl.ds(base, CE)], dst_v)
            pltpu.async_copy(tvb_hbm.at[eix_v], rows_v, sem).wait()
            pltpu.sync_copy(rows_v, acc_sh.at[dst_v], add=True)
        _chunks(n_ec, w, edge_b, active=c == 1)

        plsc.subcore_barrier()

        def dump_a(ci):
            base = ci * CM
            pltpu.sync_copy(acc_sh.at[pl.ds(base, CM)], prow_v)
            pltpu.sync_copy(prow_v, voxa_hbm.at[pl.ds(base, CM)])
        _chunks(M // CM, w, dump_a, active=c == 0)

        def dump_b(ci):
            base = ci * CM
            pltpu.sync_copy(acc_sh.at[pl.ds(base, CM)], prow_v)
            pltpu.sync_copy(prow_v, voxb_hbm.at[pl.ds(base, CM)])
        _chunks(M // CM, w, dump_b, active=c == 1)

        plsc.subcore_barrier()

        def point_a(ci):
            base = ci * CN
            pltpu.sync_copy(p2v_hbm.at[pl.ds(base, CN)], pix_v)
            pltpu.async_copy(voxa_hbm.at[pix_v], prow_v, sem).wait()
            pltpu.sync_copy(prow_v, outa_hbm.at[pl.ds(base, CN)])
        _chunks(N // CN, w, point_a, active=c == 0)

        def point_b(ci):
            base = ci * CN
            pltpu.sync_copy(p2v_hbm.at[pl.ds(base, CN)], pix_v)
            pltpu.async_copy(voxb_hbm.at[pix_v], prow_v, sem).wait()
            pltpu.sync_copy(prow_v, outb_hbm.at[pl.ds(base, CN)])
        _chunks(N // CN, w, point_b, active=c == 1)

    return body(tv_a, tv_b, eidx, dst, p2v, zrow)[:2]


def _sc_voxelize_rh(rh, p2v, zrow):
    @functools.partial(
        pl.kernel,
        out_type=[
            jax.ShapeDtypeStruct((M, H), f32),
            jax.ShapeDtypeStruct((M, H), f32),
        ],
        mesh=_mesh(),
        scratch_types=[
            pltpu.VMEM_SHARED((M, H), f32),
            pltpu.VMEM((CN,), i32),
            pltpu.VMEM((CN, H), f32),
            pltpu.SemaphoreType.DMA,
        ],
    )
    def body(rh_hbm, p2v_hbm, zrow_hbm, sum0_hbm, sum1_hbm,
             acc_sh, idx_v, rows_v, sem):
        c = lax.axis_index("c")
        w = lax.axis_index("s")

        pltpu.sync_copy(zrow_hbm, rows_v)

        def zero_chunk(ci):
            pltpu.sync_copy(rows_v, acc_sh.at[pl.ds(ci * CM, CM)])
        _chunks(M // CM, w, zero_chunk)
        plsc.subcore_barrier()

        pbase0 = c * (N // 2)

        def scatter_chunk(ci):
            base = pbase0 + ci * CN
            pltpu.sync_copy(p2v_hbm.at[pl.ds(base, CN)], idx_v)
            pltpu.sync_copy(rh_hbm.at[pl.ds(base, CN)], rows_v)
            pltpu.sync_copy(rows_v, acc_sh.at[idx_v], add=True)
        _chunks((N // 2) // CN, w, scatter_chunk)

        plsc.subcore_barrier()

        def dump_0(ci):
            base = ci * CM
            pltpu.sync_copy(acc_sh.at[pl.ds(base, CM)], rows_v)
            pltpu.sync_copy(rows_v, sum0_hbm.at[pl.ds(base, CM)])
        _chunks(M // CM, w, dump_0, active=c == 0)

        def dump_1(ci):
            base = ci * CM
            pltpu.sync_copy(acc_sh.at[pl.ds(base, CM)], rows_v)
            pltpu.sync_copy(rows_v, sum1_hbm.at[pl.ds(base, CM)])
        _chunks(M // CM, w, dump_1, active=c == 1)

    return body(rh, p2v, zrow)


MB = 2000
NB = 4000


def _tc_einsum_zr(hsum, xsum, cnt, Wz_c, Wr_c):
    def kern(hs_ref, xs_ref, cnt_ref, wz_ref, wr_ref, tvz_ref, tvr_ref):
        r = 1.0 / jnp.maximum(cnt_ref[:, 0:1], 1.0)
        vhx = jnp.concatenate([hs_ref[...] * r, xs_ref[...] * r], axis=1)
        tvz_ref[0] = jnp.dot(vhx, wz_ref[0], preferred_element_type=f32)
        tvr_ref[0] = jnp.dot(vhx, wr_ref[0], preferred_element_type=f32)

    grid = (M // MB, K)
    return pl.pallas_call(
        kern,
        grid=grid,
        in_specs=[
            pl.BlockSpec((MB, H), lambda m, k: (m, 0)),
            pl.BlockSpec((MB, H), lambda m, k: (m, 0)),
            pl.BlockSpec((MB, 16), lambda m, k: (m, 0)),
            pl.BlockSpec((1, 2 * H, H), lambda m, k: (k, 0, 0)),
            pl.BlockSpec((1, 2 * H, H), lambda m, k: (k, 0, 0)),
        ],
        out_specs=[
            pl.BlockSpec((1, MB, H), lambda m, k: (k, m, 0)),
            pl.BlockSpec((1, MB, H), lambda m, k: (k, m, 0)),
        ],
        out_shape=[
            jax.ShapeDtypeStruct((K, M, H), f32),
            jax.ShapeDtypeStruct((K, M, H), f32),
        ],
    )(hsum, xsum, cnt, Wz_c, Wr_c)


def _tc_einsum_q(rh0, rh1, xsum, cnt, Wq_c):
    def kern(a_ref, b_ref, xs_ref, cnt_ref, wq_ref, tvq_ref):
        r = 1.0 / jnp.maximum(cnt_ref[:, 0:1], 1.0)
        vhx = jnp.concatenate(
            [(a_ref[...] + b_ref[...]) * r, xs_ref[...] * r], axis=1)
        tvq_ref[0] = jnp.dot(vhx, wq_ref[0], preferred_element_type=f32)

    grid = (M // MB, K)
    return pl.pallas_call(
        kern,
        grid=grid,
        in_specs=[
            pl.BlockSpec((MB, H), lambda m, k: (m, 0)),
            pl.BlockSpec((MB, H), lambda m, k: (m, 0)),
            pl.BlockSpec((MB, H), lambda m, k: (m, 0)),
            pl.BlockSpec((MB, 16), lambda m, k: (m, 0)),
            pl.BlockSpec((1, 2 * H, H), lambda m, k: (k, 0, 0)),
        ],
        out_specs=pl.BlockSpec((1, MB, H), lambda m, k: (k, m, 0)),
        out_shape=jax.ShapeDtypeStruct((K, M, H), f32),
    )(rh0, rh1, xsum, cnt, Wq_c)


def _tc_gates(h_F, x_F, ozf, orf, Wz_l, bz_l, Wr_l, br_l, Wq_l, bq_l):
    def kern(h_ref, x_ref, oz_ref, or_ref, wz_ref, bz_ref, wr_ref, br_ref,
             wq_ref, bq_ref, z_ref, rh_ref, lq_ref):
        h = h_ref[...]
        x = x_ref[...]
        lin_z = (jnp.dot(h, wz_ref[:H], preferred_element_type=f32)
                 + jnp.dot(x, wz_ref[H:], preferred_element_type=f32)
                 + bz_ref[...])
        lin_r = (jnp.dot(h, wr_ref[:H], preferred_element_type=f32)
                 + jnp.dot(x, wr_ref[H:], preferred_element_type=f32)
                 + br_ref[...])
        z = jax.nn.sigmoid(oz_ref[...] + lin_z)
        r = jax.nn.sigmoid(or_ref[...] + lin_r)
        rh = r * h
        lq = (jnp.dot(rh, wq_ref[:H], preferred_element_type=f32)
              + jnp.dot(x, wq_ref[H:], preferred_element_type=f32)
              + bq_ref[...])
        z_ref[...] = z
        rh_ref[...] = rh
        lq_ref[...] = lq

    grid = (N // NB,)
    row = pl.BlockSpec((NB, H), lambda n: (n, 0))
    wspec = pl.BlockSpec((2 * H, H), lambda n: (0, 0))
    bspec = pl.BlockSpec((1, H), lambda n: (0, 0))
    return pl.pallas_call(
        kern,
        grid=grid,
        in_specs=[row, row, row, row, wspec, bspec, wspec, bspec, wspec, bspec],
        out_specs=[row, row, row],
        out_shape=[
            jax.ShapeDtypeStruct((N, H), f32),
            jax.ShapeDtypeStruct((N, H), f32),
            jax.ShapeDtypeStruct((N, H), f32),
        ],
    )(h_F, x_F, ozf, orf, Wz_l, bz_l.reshape(1, H), Wr_l, br_l.reshape(1, H),
      Wq_l, bq_l.reshape(1, H))


def _tc_final(h_F, z, q0, q1, lq):
    def kern(h_ref, z_ref, q0_ref, q1_ref, lq_ref, out_ref):
        q = jnp.tanh(q0_ref[...] + q1_ref[...] + lq_ref[...])
        z = z_ref[...]
        out_ref[...] = (1.0 - z) * h_ref[...] + z * q

    grid = (N // NB,)
    row = pl.BlockSpec((NB, H), lambda n: (n, 0))
    return pl.pallas_call(
        kern,
        grid=grid,
        in_specs=[row, row, row, row, row],
        out_specs=row,
        out_shape=jax.ShapeDtypeStruct((N, H), f32),
    )(h_F, z, q0, q1, lq)


def kernel(h_F, x_F, point2voxel, edge_index, edge_kernel,
           Wz_c, Wz_l, bz_l, Wr_c, Wr_l, br_l, Wq_c, Wq_l, bq_l):
    src = edge_index[0]
    dst = edge_index[1]
    eidx = edge_kernel * M + src

    zrowN = jnp.zeros((CN, H), f32)
    zrowM = jnp.zeros((CM, H), f32)
    onesN = jnp.ones((CN, H), f32)

    hsum, xsum, cnt0, cnt1 = _sc_voxelize(h_F, x_F, point2voxel,
                                          zrowN, onesN)
    cnt = cnt0[:, :16] + cnt1[:, :16]
    tv_z, tv_r = _tc_einsum_zr(hsum, xsum, cnt, Wz_c, Wr_c)
    out_zF, out_rF = _sc_edge_pass(False, tv_z.reshape(K * M, H),
                                   tv_r.reshape(K * M, H),
                                   eidx, dst, point2voxel, zrowM)
    z, rh, lq = _tc_gates(h_F, x_F, out_zF, out_rF,
                          Wz_l, bz_l, Wr_l, br_l, Wq_l, bq_l)
    rh0, rh1 = _sc_voxelize_rh(rh, point2voxel, zrowN)
    tv_q = _tc_einsum_q(rh0, rh1, xsum, cnt, Wq_c)
    q0, q1 = _sc_edge_pass(True, tv_q.reshape(K * M, H),
                           tv_q.reshape(K * M, H),
                           eidx, dst, point2voxel, zrowM)
    h_new = _tc_final(h_F, z, q0, q1, lq)
    return h_new


# speedup vs baseline: 1.2512x; 1.0687x over previous
import functools

import jax
import jax.numpy as jnp
from jax import lax
from jax.experimental import pallas as pl
from jax.experimental.pallas import tpu as pltpu
from jax.experimental.pallas import tpu_sc as plsc

N = 100000
M = 10000
E = 160000
K = 27
H = 128

NC, NS = 2, 16

CN = 80
CE = 128
CM = 80

f32 = jnp.float32
i32 = jnp.int32


def _mesh():
    return plsc.VectorSubcoreMesh(
        core_axis_name="c", subcore_axis_name="s", num_cores=NC, num_subcores=NS
    )


def _chunks(nchunks, w, body, active=None):
    n = (nchunks - w + NS - 1) // NS
    if active is not None:
        n = jnp.where(active, n, 0)

    def f(i, carry):
        body(w + i * NS)
        return carry
    lax.fori_loop(0, n, f, 0)


def _sc_voxelize(h_F, x_F, p2v, zrow, onesrow):
    @functools.partial(
        pl.kernel,
        out_type=[
            jax.ShapeDtypeStruct((M, H), f32),
            jax.ShapeDtypeStruct((M, H), f32),
            jax.ShapeDtypeStruct((M, H), f32),
            jax.ShapeDtypeStruct((M, H), f32),
        ],
        mesh=_mesh(),
        scratch_types=[
            pltpu.VMEM_SHARED((M, H), f32),
            pltpu.VMEM((CN,), i32),
            pltpu.VMEM((CN, H), f32),
            pltpu.VMEM((CN, H), f32),
            pltpu.SemaphoreType.DMA,
        ],
    )
    def body(h_hbm, x_hbm, p2v_hbm, zrow_hbm, onesrow_hbm,
             hsum_hbm, xsum_hbm, cnt0_hbm, cnt1_hbm,
             acc_sh, idx_v, rows_v, ones_v, sem):
        c = lax.axis_index("c")
        w = lax.axis_index("s")

        pltpu.sync_copy(zrow_hbm, rows_v)
        pltpu.sync_copy(onesrow_hbm, ones_v)

        def zero_chunk(ci):
            pltpu.sync_copy(rows_v, acc_sh.at[pl.ds(ci * CM, CM)])
        _chunks(M // CM, w, zero_chunk)
        plsc.subcore_barrier()

        def scatter_h(ci):
            base = ci * CN
            pltpu.sync_copy(p2v_hbm.at[pl.ds(base, CN)], idx_v)
            pltpu.sync_copy(h_hbm.at[pl.ds(base, CN)], rows_v)
            pltpu.sync_copy(rows_v, acc_sh.at[idx_v], add=True)
        _chunks(N // CN, w, scatter_h, active=c == 0)

        def scatter_x(ci):
            base = ci * CN
            pltpu.sync_copy(p2v_hbm.at[pl.ds(base, CN)], idx_v)
            pltpu.sync_copy(x_hbm.at[pl.ds(base, CN)], rows_v)
            pltpu.sync_copy(rows_v, acc_sh.at[idx_v], add=True)
        _chunks(N // CN, w, scatter_x, active=c == 1)

        plsc.subcore_barrier()

        def dump_h(ci):
            base = ci * CM
            pltpu.sync_copy(acc_sh.at[pl.ds(base, CM)], rows_v)
            pltpu.sync_copy(rows_v, hsum_hbm.at[pl.ds(base, CM)])
        _chunks(M // CM, w, dump_h, active=c == 0)

        def dump_x(ci):
            base = ci * CM
            pltpu.sync_copy(acc_sh.at[pl.ds(base, CM)], rows_v)
            pltpu.sync_copy(rows_v, xsum_hbm.at[pl.ds(base, CM)])
        _chunks(M // CM, w, dump_x, active=c == 1)

        plsc.subcore_barrier()

        pltpu.sync_copy(zrow_hbm, rows_v)

        def zero2_chunk(ci):
            pltpu.sync_copy(rows_v, acc_sh.at[pl.ds(ci * CM, CM)])
        _chunks(M // CM, w, zero2_chunk)
        plsc.subcore_barrier()

        pbase0 = c * (N // 2)

        def scatter_ones(ci):
            base = pbase0 + ci * CN
            pltpu.sync_copy(p2v_hbm.at[pl.ds(base, CN)], idx_v)
            pltpu.sync_copy(ones_v, acc_sh.at[idx_v], add=True)
        _chunks((N // 2) // CN, w, scatter_ones)

        plsc.subcore_barrier()

        def dump_c0(ci):
            base = ci * CM
            pltpu.sync_copy(acc_sh.at[pl.ds(base, CM)], rows_v)
            pltpu.sync_copy(rows_v, cnt0_hbm.at[pl.ds(base, CM)])
        _chunks(M // CM, w, dump_c0, active=c == 0)

        def dump_c1(ci):
            base = ci * CM
            pltpu.sync_copy(acc_sh.at[pl.ds(base, CM)], rows_v)
            pltpu.sync_copy(rows_v, cnt1_hbm.at[pl.ds(base, CM)])
        _chunks(M // CM, w, dump_c1, active=c == 1)

    return body(h_F, x_F, p2v, zrow, onesrow)


def _sc_edge_pass(split_edges, tv_a, tv_b, eidx, dst, p2v, zrow):
    n_ec = (E // 2 if split_edges else E) // CE

    @functools.partial(
        pl.kernel,
        out_type=[
            jax.ShapeDtypeStruct((N, H), f32),
            jax.ShapeDtypeStruct((N, H), f32),
            jax.ShapeDtypeStruct((M, H), f32),
            jax.ShapeDtypeStruct((M, H), f32),
        ],
        mesh=_mesh(),
        scratch_types=[
            pltpu.VMEM_SHARED((M, H), f32),
            pltpu.VMEM((CE,), i32),
            pltpu.VMEM((CE,), i32),
            pltpu.VMEM((CE, H), f32),
            pltpu.VMEM((CN,), i32),
            pltpu.VMEM((CN, H), f32),
            pltpu.SemaphoreType.DMA,
        ],
    )
    def body(tva_hbm, tvb_hbm, eidx_hbm, dst_hbm, p2v_hbm, zrow_hbm,
             outa_hbm, outb_hbm, voxa_hbm, voxb_hbm,
             acc_sh, eix_v, dst_v, rows_v, pix_v, prow_v, sem):
        c = lax.axis_index("c")
        w = lax.axis_index("s")

        pltpu.sync_copy(zrow_hbm, prow_v)

        def zero_chunk(ci):
            pltpu.sync_copy(prow_v, acc_sh.at[pl.ds(ci * CM, CM)])
        _chunks(M // CM, w, zero_chunk)
        plsc.subcore_barrier()

        ebase0 = c * (E // 2) if split_edges else 0

        def edge_a(ci):
            base = ebase0 + ci * CE
            pltpu.sync_copy(eidx_hbm.at[pl.ds(base, CE)], eix_v)
            pltpu.sync_copy(dst_hbm.at[pl.ds(base, CE)], dst_v)
            pltpu.async_copy(tva_hbm.at[eix_v], rows_v, sem).wait()
            pltpu.sync_copy(rows_v, acc_sh.at[dst_v], add=True)
        _chunks(n_ec, w, edge_a, active=c == 0)

        def edge_b(ci):
            base = ebase0 + ci * CE
            pltpu.sync_copy(eidx_hbm.at[pl.ds(base, CE)], eix_v)
            pltpu.sync_copy(dst_hbm.at[pl.ds(base, CE)], dst_v)
            pltpu.async_copy(tvb_hbm.at[eix_v], rows_v, sem).wait()
            pltpu.sync_copy(rows_v, acc_sh.at[dst_v], add=True)
        _chunks(n_ec, w, edge_b, active=c == 1)

        plsc.subcore_barrier()

        def dump_a(ci):
            base = ci * CM
            pltpu.sync_copy(acc_sh.at[pl.ds(base, CM)], prow_v)
            pltpu.sync_copy(prow_v, voxa_hbm.at[pl.ds(base, CM)])
        _chunks(M // CM, w, dump_a, active=c == 0)

        def dump_b(ci):
            base = ci * CM
            pltpu.sync_copy(acc_sh.at[pl.ds(base, CM)], prow_v)
            pltpu.sync_copy(prow_v, voxb_hbm.at[pl.ds(base, CM)])
        _chunks(M // CM, w, dump_b, active=c == 1)

        plsc.subcore_barrier()

        def point_a(ci):
            base = ci * CN
            pltpu.sync_copy(p2v_hbm.at[pl.ds(base, CN)], pix_v)
            pltpu.async_copy(voxa_hbm.at[pix_v], prow_v, sem).wait()
            pltpu.sync_copy(prow_v, outa_hbm.at[pl.ds(base, CN)])
        _chunks(N // CN, w, point_a, active=c == 0)

        def point_b(ci):
            base = ci * CN
            pltpu.sync_copy(p2v_hbm.at[pl.ds(base, CN)], pix_v)
            pltpu.async_copy(voxb_hbm.at[pix_v], prow_v, sem).wait()
            pltpu.sync_copy(prow_v, outb_hbm.at[pl.ds(base, CN)])
        _chunks(N // CN, w, point_b, active=c == 1)

    return body(tv_a, tv_b, eidx, dst, p2v, zrow)[:2]


def _sc_voxelize_rh(rh, p2v, zrow):
    @functools.partial(
        pl.kernel,
        out_type=[
            jax.ShapeDtypeStruct((M, H), f32),
            jax.ShapeDtypeStruct((M, H), f32),
        ],
        mesh=_mesh(),
        scratch_types=[
            pltpu.VMEM_SHARED((M, H), f32),
            pltpu.VMEM((CN,), i32),
            pltpu.VMEM((CN, H), f32),
            pltpu.SemaphoreType.DMA,
        ],
    )
    def body(rh_hbm, p2v_hbm, zrow_hbm, sum0_hbm, sum1_hbm,
             acc_sh, idx_v, rows_v, sem):
        c = lax.axis_index("c")
        w = lax.axis_index("s")

        pltpu.sync_copy(zrow_hbm, rows_v)

        def zero_chunk(ci):
            pltpu.sync_copy(rows_v, acc_sh.at[pl.ds(ci * CM, CM)])
        _chunks(M // CM, w, zero_chunk)
        plsc.subcore_barrier()

        pbase0 = c * (N // 2)

        def scatter_chunk(ci):
            base = pbase0 + ci * CN
            pltpu.sync_copy(p2v_hbm.at[pl.ds(base, CN)], idx_v)
            pltpu.sync_copy(rh_hbm.at[pl.ds(base, CN)], rows_v)
            pltpu.sync_copy(rows_v, acc_sh.at[idx_v], add=True)
        _chunks((N // 2) // CN, w, scatter_chunk)

        plsc.subcore_barrier()

        def dump_0(ci):
            base = ci * CM
            pltpu.sync_copy(acc_sh.at[pl.ds(base, CM)], rows_v)
            pltpu.sync_copy(rows_v, sum0_hbm.at[pl.ds(base, CM)])
        _chunks(M // CM, w, dump_0, active=c == 0)

        def dump_1(ci):
            base = ci * CM
            pltpu.sync_copy(acc_sh.at[pl.ds(base, CM)], rows_v)
            pltpu.sync_copy(rows_v, sum1_hbm.at[pl.ds(base, CM)])
        _chunks(M // CM, w, dump_1, active=c == 1)

    return body(rh, p2v, zrow)


MB = 5000
NB = 5000


def _tc_einsum_zr(hsum, xsum, cnt, Wz_c, Wr_c):
    def kern(hs_ref, xs_ref, cnt_ref, wz_ref, wr_ref, tvz_ref, tvr_ref):
        r = 1.0 / jnp.maximum(cnt_ref[:, 0:1], 1.0)
        vhx = jnp.concatenate([hs_ref[...] * r, xs_ref[...] * r], axis=1)
        tvz_ref[0] = jnp.dot(vhx, wz_ref[0], preferred_element_type=f32)
        tvr_ref[0] = jnp.dot(vhx, wr_ref[0], preferred_element_type=f32)

    grid = (M // MB, K)
    return pl.pallas_call(
        kern,
        grid=grid,
        in_specs=[
            pl.BlockSpec((MB, H), lambda m, k: (m, 0)),
            pl.BlockSpec((MB, H), lambda m, k: (m, 0)),
            pl.BlockSpec((MB, 16), lambda m, k: (m, 0)),
            pl.BlockSpec((1, 2 * H, H), lambda m, k: (k, 0, 0)),
            pl.BlockSpec((1, 2 * H, H), lambda m, k: (k, 0, 0)),
        ],
        out_specs=[
            pl.BlockSpec((1, MB, H), lambda m, k: (k, m, 0)),
            pl.BlockSpec((1, MB, H), lambda m, k: (k, m, 0)),
        ],
        out_shape=[
            jax.ShapeDtypeStruct((K, M, H), f32),
            jax.ShapeDtypeStruct((K, M, H), f32),
        ],
    )(hsum, xsum, cnt, Wz_c, Wr_c)


def _tc_einsum_q(rh0, rh1, xsum, cnt, Wq_c):
    def kern(a_ref, b_ref, xs_ref, cnt_ref, wq_ref, tvq_ref):
        r = 1.0 / jnp.maximum(cnt_ref[:, 0:1], 1.0)
        vhx = jnp.concatenate(
            [(a_ref[...] + b_ref[...]) * r, xs_ref[...] * r], axis=1)
        tvq_ref[0] = jnp.dot(vhx, wq_ref[0], preferred_element_type=f32)

    grid = (M // MB, K)
    return pl.pallas_call(
        kern,
        grid=grid,
        in_specs=[
            pl.BlockSpec((MB, H), lambda m, k: (m, 0)),
            pl.BlockSpec((MB, H), lambda m, k: (m, 0)),
            pl.BlockSpec((MB, H), lambda m, k: (m, 0)),
            pl.BlockSpec((MB, 16), lambda m, k: (m, 0)),
            pl.BlockSpec((1, 2 * H, H), lambda m, k: (k, 0, 0)),
        ],
        out_specs=pl.BlockSpec((1, MB, H), lambda m, k: (k, m, 0)),
        out_shape=jax.ShapeDtypeStruct((K, M, H), f32),
    )(rh0, rh1, xsum, cnt, Wq_c)


def _tc_gates(h_F, x_F, ozf, orf, Wz_l, bz_l, Wr_l, br_l, Wq_l, bq_l):
    def kern(h_ref, x_ref, oz_ref, or_ref, wz_ref, bz_ref, wr_ref, br_ref,
             wq_ref, bq_ref, z_ref, rh_ref, lq_ref):
        h = h_ref[...]
        x = x_ref[...]
        lin_z = (jnp.dot(h, wz_ref[:H], preferred_element_type=f32)
                 + jnp.dot(x, wz_ref[H:], preferred_element_type=f32)
                 + bz_ref[...])
        lin_r = (jnp.dot(h, wr_ref[:H], preferred_element_type=f32)
                 + jnp.dot(x, wr_ref[H:], preferred_element_type=f32)
                 + br_ref[...])
        z = jax.nn.sigmoid(oz_ref[...] + lin_z)
        r = jax.nn.sigmoid(or_ref[...] + lin_r)
        rh = r * h
        lq = (jnp.dot(rh, wq_ref[:H], preferred_element_type=f32)
              + jnp.dot(x, wq_ref[H:], preferred_element_type=f32)
              + bq_ref[...])
        z_ref[...] = z
        rh_ref[...] = rh
        lq_ref[...] = lq

    grid = (N // NB,)
    row = pl.BlockSpec((NB, H), lambda n: (n, 0))
    wspec = pl.BlockSpec((2 * H, H), lambda n: (0, 0))
    bspec = pl.BlockSpec((1, H), lambda n: (0, 0))
    return pl.pallas_call(
        kern,
        grid=grid,
        in_specs=[row, row, row, row, wspec, bspec, wspec, bspec, wspec, bspec],
        out_specs=[row, row, row],
        out_shape=[
            jax.ShapeDtypeStruct((N, H), f32),
            jax.ShapeDtypeStruct((N, H), f32),
            jax.ShapeDtypeStruct((N, H), f32),
        ],
    )(h_F, x_F, ozf, orf, Wz_l, bz_l.reshape(1, H), Wr_l, br_l.reshape(1, H),
      Wq_l, bq_l.reshape(1, H))


def _tc_final(h_F, z, q0, q1, lq):
    def kern(h_ref, z_ref, q0_ref, q1_ref, lq_ref, out_ref):
        q = jnp.tanh(q0_ref[...] + q1_ref[...] + lq_ref[...])
        z = z_ref[...]
        out_ref[...] = (1.0 - z) * h_ref[...] + z * q

    grid = (N // NB,)
    row = pl.BlockSpec((NB, H), lambda n: (n, 0))
    return pl.pallas_call(
        kern,
        grid=grid,
        in_specs=[row, row, row, row, row],
        out_specs=row,
        out_shape=jax.ShapeDtypeStruct((N, H), f32),
    )(h_F, z, q0, q1, lq)


def kernel(h_F, x_F, point2voxel, edge_index, edge_kernel,
           Wz_c, Wz_l, bz_l, Wr_c, Wr_l, br_l, Wq_c, Wq_l, bq_l):
    src = edge_index[0]
    dst = edge_index[1]
    eidx = edge_kernel * M + src

    zrowN = jnp.zeros((CN, H), f32)
    zrowM = jnp.zeros((CM, H), f32)
    onesN = jnp.ones((CN, H), f32)

    hsum, xsum, cnt0, cnt1 = _sc_voxelize(h_F, x_F, point2voxel,
                                          zrowN, onesN)
    cnt = cnt0[:, :16] + cnt1[:, :16]
    tv_z, tv_r = _tc_einsum_zr(hsum, xsum, cnt, Wz_c, Wr_c)
    out_zF, out_rF = _sc_edge_pass(False, tv_z.reshape(K * M, H),
                                   tv_r.reshape(K * M, H),
                                   eidx, dst, point2voxel, zrowM)
    z, rh, lq = _tc_gates(h_F, x_F, out_zF, out_rF,
                          Wz_l, bz_l, Wr_l, br_l, Wq_l, bq_l)
    rh0, rh1 = _sc_voxelize_rh(rh, point2voxel, zrowN)
    tv_q = _tc_einsum_q(rh0, rh1, xsum, cnt, Wq_c)
    q0, q1 = _sc_edge_pass(True, tv_q.reshape(K * M, H),
                           tv_q.reshape(K * M, H),
                           eidx, dst, point2voxel, zrowM)
    h_new = _tc_final(h_F, z, q0, q1, lq)
    return h_new


# speedup vs baseline: 1.2769x; 1.0206x over previous
import functools

import jax
import jax.numpy as jnp
from jax import lax
from jax.experimental import pallas as pl
from jax.experimental.pallas import tpu as pltpu
from jax.experimental.pallas import tpu_sc as plsc

N = 100000
M = 10000
E = 160000
K = 27
H = 128

NC, NS = 2, 16

CN = 80
CE = 128
CM = 80

f32 = jnp.float32
i32 = jnp.int32


def _mesh():
    return plsc.VectorSubcoreMesh(
        core_axis_name="c", subcore_axis_name="s", num_cores=NC, num_subcores=NS
    )


def _chunks(nchunks, w, body, active=None):
    n = (nchunks - w + NS - 1) // NS
    if active is not None:
        n = jnp.where(active, n, 0)

    def f(i, carry):
        body(w + i * NS)
        return carry
    lax.fori_loop(0, n, f, 0)


def _sc_voxelize(h_F, x_F, p2v, zrow, onesrow):
    @functools.partial(
        pl.kernel,
        out_type=[
            jax.ShapeDtypeStruct((M, H), f32),
            jax.ShapeDtypeStruct((M, H), f32),
            jax.ShapeDtypeStruct((M, H), f32),
            jax.ShapeDtypeStruct((M, H), f32),
        ],
        mesh=_mesh(),
        scratch_types=[
            pltpu.VMEM_SHARED((M, H), f32),
            pltpu.VMEM((CN,), i32),
            pltpu.VMEM((CN, H), f32),
            pltpu.VMEM((CN, H), f32),
            pltpu.SemaphoreType.DMA,
        ],
    )
    def body(h_hbm, x_hbm, p2v_hbm, zrow_hbm, onesrow_hbm,
             hsum_hbm, xsum_hbm, cnt0_hbm, cnt1_hbm,
             acc_sh, idx_v, rows_v, ones_v, sem):
        c = lax.axis_index("c")
        w = lax.axis_index("s")

        pltpu.sync_copy(zrow_hbm, rows_v)
        pltpu.sync_copy(onesrow_hbm, ones_v)

        def zero_chunk(ci):
            pltpu.sync_copy(rows_v, acc_sh.at[pl.ds(ci * CM, CM)])
        _chunks(M // CM, w, zero_chunk)
        plsc.subcore_barrier()

        def scatter_h(ci):
            base = ci * CN
            pltpu.sync_copy(p2v_hbm.at[pl.ds(base, CN)], idx_v)
            pltpu.sync_copy(h_hbm.at[pl.ds(base, CN)], rows_v)
            pltpu.sync_copy(rows_v, acc_sh.at[idx_v], add=True)
        _chunks(N // CN, w, scatter_h, active=c == 0)

        def scatter_x(ci):
            base = ci * CN
            pltpu.sync_copy(p2v_hbm.at[pl.ds(base, CN)], idx_v)
            pltpu.sync_copy(x_hbm.at[pl.ds(base, CN)], rows_v)
            pltpu.sync_copy(rows_v, acc_sh.at[idx_v], add=True)
        _chunks(N // CN, w, scatter_x, active=c == 1)

        plsc.subcore_barrier()

        def dump_h(ci):
            base = ci * CM
            pltpu.sync_copy(acc_sh.at[pl.ds(base, CM)], rows_v)
            pltpu.sync_copy(rows_v, hsum_hbm.at[pl.ds(base, CM)])
        _chunks(M // CM, w, dump_h, active=c == 0)

        def dump_x(ci):
            base = ci * CM
            pltpu.sync_copy(acc_sh.at[pl.ds(base, CM)], rows_v)
            pltpu.sync_copy(rows_v, xsum_hbm.at[pl.ds(base, CM)])
        _chunks(M // CM, w, dump_x, active=c == 1)

        plsc.subcore_barrier()

        pltpu.sync_copy(zrow_hbm, rows_v)

        def zero2_chunk(ci):
            pltpu.sync_copy(rows_v, acc_sh.at[pl.ds(ci * CM, CM)])
        _chunks(M // CM, w, zero2_chunk)
        plsc.subcore_barrier()

        pbase0 = c * (N // 2)

        def scatter_ones(ci):
            base = pbase0 + ci * CN
            pltpu.sync_copy(p2v_hbm.at[pl.ds(base, CN)], idx_v)
            pltpu.sync_copy(ones_v, acc_sh.at[idx_v], add=True)
        _chunks((N // 2) // CN, w, scatter_ones)

        plsc.subcore_barrier()

        def dump_c0(ci):
            base = ci * CM
            pltpu.sync_copy(acc_sh.at[pl.ds(base, CM)], rows_v)
            pltpu.sync_copy(rows_v, cnt0_hbm.at[pl.ds(base, CM)])
        _chunks(M // CM, w, dump_c0, active=c == 0)

        def dump_c1(ci):
            base = ci * CM
            pltpu.sync_copy(acc_sh.at[pl.ds(base, CM)], rows_v)
            pltpu.sync_copy(rows_v, cnt1_hbm.at[pl.ds(base, CM)])
        _chunks(M // CM, w, dump_c1, active=c == 1)

    return body(h_F, x_F, p2v, zrow, onesrow)


def _sc_edge_pass(split_edges, tv_a, tv_b, eidx, dst, p2v, zrow):
    n_ec = (E // 2 if split_edges else E) // CE

    @functools.partial(
        pl.kernel,
        out_type=[
            jax.ShapeDtypeStruct((N, H), f32),
            jax.ShapeDtypeStruct((N, H), f32),
            jax.ShapeDtypeStruct((M, H), f32),
            jax.ShapeDtypeStruct((M, H), f32),
        ],
        mesh=_mesh(),
        scratch_types=[
            pltpu.VMEM_SHARED((M, H), f32),
            pltpu.VMEM((CE,), i32),
            pltpu.VMEM((CE,), i32),
            pltpu.VMEM((CE, H), f32),
            pltpu.VMEM((CN,), i32),
            pltpu.VMEM((CN, H), f32),
            pltpu.SemaphoreType.DMA,
        ],
    )
    def body(tva_hbm, tvb_hbm, eidx_hbm, dst_hbm, p2v_hbm, zrow_hbm,
             outa_hbm, outb_hbm, voxa_hbm, voxb_hbm,
             acc_sh, eix_v, dst_v, rows_v, pix_v, prow_v, sem):
        c = lax.axis_index("c")
        w = lax.axis_index("s")

        pltpu.sync_copy(zrow_hbm, prow_v)

        def zero_chunk(ci):
            pltpu.sync_copy(prow_v, acc_sh.at[pl.ds(ci * CM, CM)])
        _chunks(M // CM, w, zero_chunk)
        plsc.subcore_barrier()

        ebase0 = c * (E // 2) if split_edges else 0

        def edge_a(ci):
            base = ebase0 + ci * CE
            pltpu.sync_copy(eidx_hbm.at[pl.ds(base, CE)], eix_v)
            pltpu.sync_copy(dst_hbm.at[pl.ds(base, CE)], dst_v)
            pltpu.async_copy(tva_hbm.at[eix_v], rows_v, sem).wait()
            pltpu.sync_copy(rows_v, acc_sh.at[dst_v], add=True)
        _chunks(n_ec, w, edge_a, active=c == 0)

        def edge_b(ci):
            base = ebase0 + ci * CE
            pltpu.sync_copy(eidx_hbm.at[pl.ds(base, CE)], eix_v)
            pltpu.sync_copy(dst_hbm.at[pl.ds(base, CE)], dst_v)
            pltpu.async_copy(tvb_hbm.at[eix_v], rows_v, sem).wait()
            pltpu.sync_copy(rows_v, acc_sh.at[dst_v], add=True)
        _chunks(n_ec, w, edge_b, active=c == 1)

        plsc.subcore_barrier()

        def dump_a(ci):
            base = ci * CM
            pltpu.sync_copy(acc_sh.at[pl.ds(base, CM)], prow_v)
            pltpu.sync_copy(prow_v, voxa_hbm.at[pl.ds(base, CM)])
        _chunks(M // CM, w, dump_a, active=c == 0)

        def dump_b(ci):
            base = ci * CM
            pltpu.sync_copy(acc_sh.at[pl.ds(base, CM)], prow_v)
            pltpu.sync_copy(prow_v, voxb_hbm.at[pl.ds(base, CM)])
        _chunks(M // CM, w, dump_b, active=c == 1)

        plsc.subcore_barrier()

        def point_a(ci):
            base = ci * CN
            pltpu.sync_copy(p2v_hbm.at[pl.ds(base, CN)], pix_v)
            pltpu.async_copy(voxa_hbm.at[pix_v], prow_v, sem).wait()
            pltpu.sync_copy(prow_v, outa_hbm.at[pl.ds(base, CN)])
        _chunks(N // CN, w, point_a, active=c == 0)

        def point_b(ci):
            base = ci * CN
            pltpu.sync_copy(p2v_hbm.at[pl.ds(base, CN)], pix_v)
            pltpu.async_copy(voxb_hbm.at[pix_v], prow_v, sem).wait()
            pltpu.sync_copy(prow_v, outb_hbm.at[pl.ds(base, CN)])
        _chunks(N // CN, w, point_b, active=c == 1)

    return body(tv_a, tv_b, eidx, dst, p2v, zrow)[:2]


def _sc_voxelize_rh(rh, p2v, zrow):
    @functools.partial(
        pl.kernel,
        out_type=[
            jax.ShapeDtypeStruct((M, H), f32),
            jax.ShapeDtypeStruct((M, H), f32),
        ],
        mesh=_mesh(),
        scratch_types=[
            pltpu.VMEM_SHARED((M, H), f32),
            pltpu.VMEM((CN,), i32),
            pltpu.VMEM((CN, H), f32),
            pltpu.SemaphoreType.DMA,
        ],
    )
    def body(rh_hbm, p2v_hbm, zrow_hbm, sum0_hbm, sum1_hbm,
             acc_sh, idx_v, rows_v, sem):
        c = lax.axis_index("c")
        w = lax.axis_index("s")

        pltpu.sync_copy(zrow_hbm, rows_v)

        def zero_chunk(ci):
            pltpu.sync_copy(rows_v, acc_sh.at[pl.ds(ci * CM, CM)])
        _chunks(M // CM, w, zero_chunk)
        plsc.subcore_barrier()

        pbase0 = c * (N // 2)

        def scatter_chunk(ci):
            base = pbase0 + ci * CN
            pltpu.sync_copy(p2v_hbm.at[pl.ds(base, CN)], idx_v)
            pltpu.sync_copy(rh_hbm.at[pl.ds(base, CN)], rows_v)
            pltpu.sync_copy(rows_v, acc_sh.at[idx_v], add=True)
        _chunks((N // 2) // CN, w, scatter_chunk)

        plsc.subcore_barrier()

        def dump_0(ci):
            base = ci * CM
            pltpu.sync_copy(acc_sh.at[pl.ds(base, CM)], rows_v)
            pltpu.sync_copy(rows_v, sum0_hbm.at[pl.ds(base, CM)])
        _chunks(M // CM, w, dump_0, active=c == 0)

        def dump_1(ci):
            base = ci * CM
            pltpu.sync_copy(acc_sh.at[pl.ds(base, CM)], rows_v)
            pltpu.sync_copy(rows_v, sum1_hbm.at[pl.ds(base, CM)])
        _chunks(M // CM, w, dump_1, active=c == 1)

    return body(rh, p2v, zrow)


MB = 10000
NB = 5000


def _tc_einsum_zr(hsum, xsum, cnt, Wz_c, Wr_c):
    def kern(hs_ref, xs_ref, cnt_ref, wz_ref, wr_ref, tvz_ref, tvr_ref):
        r = 1.0 / jnp.maximum(cnt_ref[:, 0:1], 1.0)
        vhx = jnp.concatenate([hs_ref[...] * r, xs_ref[...] * r], axis=1)
        tvz_ref[0] = jnp.dot(vhx, wz_ref[0], preferred_element_type=f32)
        tvr_ref[0] = jnp.dot(vhx, wr_ref[0], preferred_element_type=f32)

    grid = (M // MB, K)
    return pl.pallas_call(
        kern,
        grid=grid,
        in_specs=[
            pl.BlockSpec((MB, H), lambda m, k: (m, 0)),
            pl.BlockSpec((MB, H), lambda m, k: (m, 0)),
            pl.BlockSpec((MB, 16), lambda m, k: (m, 0)),
            pl.BlockSpec((1, 2 * H, H), lambda m, k: (k, 0, 0)),
            pl.BlockSpec((1, 2 * H, H), lambda m, k: (k, 0, 0)),
        ],
        out_specs=[
            pl.BlockSpec((1, MB, H), lambda m, k: (k, m, 0)),
            pl.BlockSpec((1, MB, H), lambda m, k: (k, m, 0)),
        ],
        out_shape=[
            jax.ShapeDtypeStruct((K, M, H), f32),
            jax.ShapeDtypeStruct((K, M, H), f32),
        ],
    )(hsum, xsum, cnt, Wz_c, Wr_c)


def _tc_einsum_q(rh0, rh1, xsum, cnt, Wq_c):
    def kern(a_ref, b_ref, xs_ref, cnt_ref, wq_ref, tvq_ref):
        r = 1.0 / jnp.maximum(cnt_ref[:, 0:1], 1.0)
        vhx = jnp.concatenate(
            [(a_ref[...] + b_ref[...]) * r, xs_ref[...] * r], axis=1)
        tvq_ref[0] = jnp.dot(vhx, wq_ref[0], preferred_element_type=f32)

    grid = (M // MB, K)
    return pl.pallas_call(
        kern,
        grid=grid,
        in_specs=[
            pl.BlockSpec((MB, H), lambda m, k: (m, 0)),
            pl.BlockSpec((MB, H), lambda m, k: (m, 0)),
            pl.BlockSpec((MB, H), lambda m, k: (m, 0)),
            pl.BlockSpec((MB, 16), lambda m, k: (m, 0)),
            pl.BlockSpec((1, 2 * H, H), lambda m, k: (k, 0, 0)),
        ],
        out_specs=pl.BlockSpec((1, MB, H), lambda m, k: (k, m, 0)),
        out_shape=jax.ShapeDtypeStruct((K, M, H), f32),
    )(rh0, rh1, xsum, cnt, Wq_c)


def _tc_gates(h_F, x_F, ozf, orf, Wz_l, bz_l, Wr_l, br_l, Wq_l, bq_l):
    def kern(h_ref, x_ref, oz_ref, or_ref, wz_ref, bz_ref, wr_ref, br_ref,
             wq_ref, bq_ref, z_ref, rh_ref, lq_ref):
        h = h_ref[...]
        x = x_ref[...]
        lin_z = (jnp.dot(h, wz_ref[:H], preferred_element_type=f32)
                 + jnp.dot(x, wz_ref[H:], preferred_element_type=f32)
                 + bz_ref[...])
        lin_r = (jnp.dot(h, wr_ref[:H], preferred_element_type=f32)
                 + jnp.dot(x, wr_ref[H:], preferred_element_type=f32)
                 + br_ref[...])
        z = jax.nn.sigmoid(oz_ref[...] + lin_z)
        r = jax.nn.sigmoid(or_ref[...] + lin_r)
        rh = r * h
        lq = (jnp.dot(rh, wq_ref[:H], preferred_element_type=f32)
              + jnp.dot(x, wq_ref[H:], preferred_element_type=f32)
              + bq_ref[...])
        z_ref[...] = z
        rh_ref[...] = rh
        lq_ref[...] = lq

    grid = (N // NB,)
    row = pl.BlockSpec((NB, H), lambda n: (n, 0))
    wspec = pl.BlockSpec((2 * H, H), lambda n: (0, 0))
    bspec = pl.BlockSpec((1, H), lambda n: (0, 0))
    return pl.pallas_call(
        kern,
        grid=grid,
        in_specs=[row, row, row, row, wspec, bspec, wspec, bspec, wspec, bspec],
        out_specs=[row, row, row],
        out_shape=[
            jax.ShapeDtypeStruct((N, H), f32),
            jax.ShapeDtypeStruct((N, H), f32),
            jax.ShapeDtypeStruct((N, H), f32),
        ],
    )(h_F, x_F, ozf, orf, Wz_l, bz_l.reshape(1, H), Wr_l, br_l.reshape(1, H),
      Wq_l, bq_l.reshape(1, H))


def _tc_final(h_F, z, q0, q1, lq):
    def kern(h_ref, z_ref, q0_ref, q1_ref, lq_ref, out_ref):
        q = jnp.tanh(q0_ref[...] + q1_ref[...] + lq_ref[...])
        z = z_ref[...]
        out_ref[...] = (1.0 - z) * h_ref[...] + z * q

    grid = (N // NB,)
    row = pl.BlockSpec((NB, H), lambda n: (n, 0))
    return pl.pallas_call(
        kern,
        grid=grid,
        in_specs=[row, row, row, row, row],
        out_specs=row,
        out_shape=jax.ShapeDtypeStruct((N, H), f32),
    )(h_F, z, q0, q1, lq)


def kernel(h_F, x_F, point2voxel, edge_index, edge_kernel,
           Wz_c, Wz_l, bz_l, Wr_c, Wr_l, br_l, Wq_c, Wq_l, bq_l):
    src = edge_index[0]
    dst = edge_index[1]
    eidx = edge_kernel * M + src

    zrowN = jnp.zeros((CN, H), f32)
    zrowM = jnp.zeros((CM, H), f32)
    onesN = jnp.ones((CN, H), f32)

    hsum, xsum, cnt0, cnt1 = _sc_voxelize(h_F, x_F, point2voxel,
                                          zrowN, onesN)
    cnt = cnt0[:, :16] + cnt1[:, :16]
    tv_z, tv_r = _tc_einsum_zr(hsum, xsum, cnt, Wz_c, Wr_c)
    out_zF, out_rF = _sc_edge_pass(False, tv_z.reshape(K * M, H),
                                   tv_r.reshape(K * M, H),
                                   eidx, dst, point2voxel, zrowM)
    z, rh, lq = _tc_gates(h_F, x_F, out_zF, out_rF,
                          Wz_l, bz_l, Wr_l, br_l, Wq_l, bq_l)
    rh0, rh1 = _sc_voxelize_rh(rh, point2voxel, zrowN)
    tv_q = _tc_einsum_q(rh0, rh1, xsum, cnt, Wq_c)
    q0, q1 = _sc_edge_pass(True, tv_q.reshape(K * M, H),
                           tv_q.reshape(K * M, H),
                           eidx, dst, point2voxel, zrowM)
    h_new = _tc_final(h_F, z, q0, q1, lq)
    return h_new
